# baseline (speedup 1.0000x reference)
import functools

import jax
import jax.numpy as jnp
from jax import lax
from jax.experimental import pallas as pl
from jax.experimental.pallas import tpu as pltpu
from jax.experimental.pallas import tpu_sc as plsc

NUM_CATEGORIES = 1000000
EMBEDDING_DIM = 64

NC = 2
NS = 16
NW = NC * NS

B_ROWS = 4096
B_COLS = 100
LANES = 16

BPW = B_ROWS // NW

TW = 256
FULL_CHUNKS = NUM_CATEGORIES // TW
BASE_CHUNKS = FULL_CHUNKS // NW
EXTRA_CHUNKS = FULL_CHUNKS - BASE_CHUNKS * NW
TAIL_OFF = FULL_CHUNKS * TW
TAIL_W = NUM_CATEGORIES - TAIL_OFF


def _transpose_table(tbl_t, tbl_tail):
    mesh = plsc.VectorSubcoreMesh(
        core_axis_name="c", subcore_axis_name="s", num_cores=NC, num_subcores=NS
    )

    @functools.partial(
        pl.kernel,
        out_type=jax.ShapeDtypeStruct((NUM_CATEGORIES // 2, 128), jnp.float32),
        mesh=mesh,
        scratch_types=[
            pltpu.VMEM((EMBEDDING_DIM, TW), jnp.float32),
            pltpu.VMEM((EMBEDDING_DIM, TW), jnp.float32),
            pltpu.VMEM((TW // 2, 128), jnp.float32),
            pltpu.VMEM((TW // 2, 128), jnp.float32),
            pltpu.SemaphoreType.DMA,
            pltpu.SemaphoreType.DMA,
            pltpu.SemaphoreType.DMA,
            pltpu.SemaphoreType.DMA,
        ],
        compiler_params=pltpu.CompilerParams(needs_layout_passes=False),
    )
    def k(tbl_hbm, tail_hbm, out_hbm, in0, in1, o0, o1, is0, is1, os0, os1):
        wid = lax.axis_index("s") * NC + lax.axis_index("c")
        iota = lax.broadcasted_iota(jnp.int32, (LANES,), 0)
        zeros = jnp.zeros((LANES,), jnp.int32)
        diak = [(iota + kk) & 15 for kk in range(LANES)]
        c064 = (iota & 1) << 6
        riota = iota >> 1
        in_v = (in0, in1)
        out_v = (o0, o1)
        isem = (is0, is1)
        osem = (os0, os1)
        cbase = wid * BASE_CHUNKS

        def start_in(c, par):
            pltpu.async_copy(
                tbl_hbm.at[:, pl.ds(pl.multiple_of((cbase + c) * TW, TW), TW)],
                in_v[par], isem[par],
            )

        def wait_in(par):
            pltpu.make_async_copy(tbl_hbm.at[:, pl.ds(0, TW)], in_v[par],
                                  isem[par]).wait()

        def start_out(c, par):
            orow = pl.multiple_of((cbase + c) * (TW // 2), TW // 2)
            pltpu.async_copy(out_v[par], out_hbm.at[pl.ds(orow, TW // 2)],
                             osem[par])

        def wait_out(par):
            pltpu.make_async_copy(out_v[par], out_hbm.at[pl.ds(0, TW // 2)],
                                  osem[par]).wait()

        def transpose(par, width):
            def per_clb(clb):
                rowvec = riota + clb * 8
                clvec = iota + clb * LANES
                for dblk in range(EMBEDDING_DIM // LANES):
                    for kk in range(LANES):
                        dvec = diak[kk] + dblk * LANES
                        x = plsc.load_gather(in_v[par], [dvec, clvec])
                        plsc.store_scatter(out_v[par], [rowvec, c064 + dvec], x)
            pl.loop(0, width // LANES)(per_clb)

        start_in(0, 0)

        def body(t):
            for par in (0, 1):
                c = t * 2 + par

                @pl.when(c + 1 < BASE_CHUNKS)
                def _():
                    start_in(c + 1, 1 - par)

                wait_in(par)

                @pl.when(c >= 2)
                def _():
                    wait_out(par)

                transpose(par, TW)
                start_out(c, par)

        pl.loop(0, BASE_CHUNKS // 2)(body)
        wait_out(0)
        wait_out(1)

        for e in range(EXTRA_CHUNKS):
            @pl.when(wid == e)
            def _():
                off = (BASE_CHUNKS * NW + e) * TW
                pltpu.sync_copy(tbl_hbm.at[:, pl.ds(off, TW)], in_v[0])
                transpose(0, TW)
                pltpu.sync_copy(out_v[0],
                                out_hbm.at[pl.ds(off // 2, TW // 2)])

        @pl.when(wid == EXTRA_CHUNKS)
        def _():
            pltpu.sync_copy(tail_hbm, in0.at[:, pl.ds(0, 128)])
            transpose(0, 128)
            pltpu.sync_copy(
                o0.at[pl.ds(0, TAIL_W // 2)],
                out_hbm.at[pl.ds(TAIL_OFF // 2, TAIL_W // 2)],
            )

    return k(tbl_t, tbl_tail)


def _gather(pairs, idx_t):
    mesh = plsc.VectorSubcoreMesh(
        core_axis_name="c", subcore_axis_name="s", num_cores=NC, num_subcores=NS
    )

    @functools.partial(
        pl.kernel,
        out_type=jax.ShapeDtypeStruct((B_COLS, EMBEDDING_DIM, B_ROWS),
                                      jnp.float32),
        mesh=mesh,
        scratch_types=[
            pltpu.VMEM((B_COLS, BPW), jnp.int32),
            pltpu.VMEM((B_COLS, BPW), jnp.int32),
            pltpu.VMEM((B_COLS, BPW), jnp.int32),
            pltpu.VMEM((BPW, 128), jnp.float32),
            pltpu.VMEM((BPW, 128), jnp.float32),
            pltpu.VMEM((EMBEDDING_DIM, BPW), jnp.float32),
            pltpu.VMEM((EMBEDDING_DIM, BPW), jnp.float32),
            pltpu.SemaphoreType.DMA,
            pltpu.SemaphoreType.DMA,
            pltpu.SemaphoreType.DMA,
            pltpu.SemaphoreType.DMA,
        ],
        compiler_params=pltpu.CompilerParams(needs_layout_passes=False),
    )
    def k(scr_hbm, idx_hbm, out_hbm, idx_v, q_v, h_v,
          r0v, r1v, o0v, o1v, gs0, gs1, os0, os1):
        wid = lax.axis_index("s") * NC + lax.axis_index("c")
        b0 = pl.multiple_of(wid * BPW, BPW)
        iota = lax.broadcasted_iota(jnp.int32, (LANES,), 0)
        rows_v = (r0v, r1v)
        out_v = (o0v, o1v)
        gsem = (gs0, gs1)
        osem = (os0, os1)
        jvec = [jb * LANES + iota for jb in range(BPW // LANES)]
        diak = [(iota + kk) & 15 for kk in range(LANES)]
        zeros = jnp.zeros((LANES,), jnp.int32)

        pltpu.sync_copy(idx_hbm.at[:, pl.ds(b0, BPW)], idx_v)

        def prep(c):
            for j in range(BPW // LANES):
                iv = idx_v[c, pl.ds(j * LANES, LANES)]
                q_v[c, pl.ds(j * LANES, LANES)] = iv >> 1
                h_v[c, pl.ds(j * LANES, LANES)] = (iv & 1) << 6
        pl.loop(0, B_COLS)(prep)

        def start_gather(c, par):
            pltpu.async_copy(scr_hbm.at[q_v.at[c]], rows_v[par], gsem[par])

        def wait_gather(c, par):
            pltpu.make_async_copy(scr_hbm.at[q_v.at[c]], rows_v[par],
                                  gsem[par]).wait()

        def start_out(c, par):
            pltpu.async_copy(out_v[par], out_hbm.at[c, :, pl.ds(b0, BPW)],
                             osem[par])

        def wait_out(c, par):
            pltpu.make_async_copy(out_v[par], out_hbm.at[c, :, pl.ds(b0, BPW)],
                                  osem[par]).wait()

        def transpose(c, par):
            for jb in range(BPW // LANES):
                hvec = h_v[c, pl.ds(jb * LANES, LANES)]

                def per_dblk(dblk):
                    dbase = dblk * LANES
                    for kk in range(LANES):
                        dvec = diak[kk] + dbase
                        x = plsc.load_gather(rows_v[par], [jvec[jb], hvec + dvec])
                        plsc.store_scatter(out_v[par], [dvec, jvec[jb]], x)
                pl.loop(0, EMBEDDING_DIM // LANES)(per_dblk)

        start_gather(0, 0)

        def body(t):
            for par in (0, 1):
                c = t * 2 + par

                @pl.when(c + 1 < B_COLS)
                def _():
                    start_gather(c + 1, 1 - par)

                wait_gather(c, par)

                @pl.when(c >= 2)
                def _():
                    wait_out(c - 2, par)

                transpose(c, par)
                start_out(c, par)

        pl.loop(0, B_COLS // 2)(body)
        wait_out(B_COLS - 2, 0)
        wait_out(B_COLS - 1, 1)

    return k(pairs, idx_t)


def kernel(category_indices, embedding_weight):
    idx_t = category_indices.astype(jnp.int32).T
    tbl_t = embedding_weight.T
    tbl_tail = lax.slice(tbl_t, (0, TAIL_OFF), (EMBEDDING_DIM, NUM_CATEGORIES))
    tbl_tail = jnp.pad(tbl_tail, ((0, 0), (0, 128 - TAIL_W)))
    pairs = _transpose_table(tbl_t, tbl_tail)
    out_t = _gather(pairs, idx_t)
    return out_t.transpose(2, 0, 1)


# speedup vs baseline: 1.2922x; 1.2922x over previous
import functools

import jax
import jax.numpy as jnp
from jax import lax
from jax.experimental import pallas as pl
from jax.experimental.pallas import tpu as pltpu
from jax.experimental.pallas import tpu_sc as plsc

NUM_CATEGORIES = 1000000
EMBEDDING_DIM = 64

NC = 2
NS = 16
NW = NC * NS

B_ROWS = 4096
B_COLS = 100
LANES = 16

BPW = B_ROWS // NW

def _gather(pairs, idx_t):
    mesh = plsc.VectorSubcoreMesh(
        core_axis_name="c", subcore_axis_name="s", num_cores=NC, num_subcores=NS
    )

    @functools.partial(
        pl.kernel,
        out_type=jax.ShapeDtypeStruct((B_COLS, EMBEDDING_DIM, B_ROWS),
                                      jnp.float32),
        mesh=mesh,
        scratch_types=[
            pltpu.VMEM((B_COLS, BPW), jnp.int32),
            pltpu.VMEM((B_COLS, BPW), jnp.int32),
            pltpu.VMEM((B_COLS, BPW), jnp.int32),
            pltpu.VMEM((BPW, 128), jnp.float32),
            pltpu.VMEM((BPW, 128), jnp.float32),
            pltpu.VMEM((EMBEDDING_DIM, BPW), jnp.float32),
            pltpu.VMEM((EMBEDDING_DIM, BPW), jnp.float32),
            pltpu.SemaphoreType.DMA,
            pltpu.SemaphoreType.DMA,
            pltpu.SemaphoreType.DMA,
            pltpu.SemaphoreType.DMA,
        ],
        compiler_params=pltpu.CompilerParams(needs_layout_passes=False),
    )
    def k(scr_hbm, idx_hbm, out_hbm, idx_v, q_v, h_v,
          r0v, r1v, o0v, o1v, gs0, gs1, os0, os1):
        wid = lax.axis_index("s") * NC + lax.axis_index("c")
        b0 = pl.multiple_of(wid * BPW, BPW)
        iota = lax.broadcasted_iota(jnp.int32, (LANES,), 0)
        rows_v = (r0v, r1v)
        out_v = (o0v, o1v)
        gsem = (gs0, gs1)
        osem = (os0, os1)
        jvec = [jb * LANES + iota for jb in range(BPW // LANES)]
        diak = [(iota + kk) & 15 for kk in range(LANES)]
        zeros = jnp.zeros((LANES,), jnp.int32)

        pltpu.sync_copy(idx_hbm.at[:, pl.ds(b0, BPW)], idx_v)

        def prep(c):
            for j in range(BPW // LANES):
                iv = idx_v[c, pl.ds(j * LANES, LANES)]
                q_v[c, pl.ds(j * LANES, LANES)] = iv >> 1
                h_v[c, pl.ds(j * LANES, LANES)] = (iv & 1) << 6
        pl.loop(0, B_COLS)(prep)

        def start_gather(c, par):
            pltpu.async_copy(scr_hbm.at[q_v.at[c]], rows_v[par], gsem[par])

        def wait_gather(c, par):
            pltpu.make_async_copy(scr_hbm.at[q_v.at[c]], rows_v[par],
                                  gsem[par]).wait()

        def start_out(c, par):
            pltpu.async_copy(out_v[par], out_hbm.at[c, :, pl.ds(b0, BPW)],
                             osem[par])

        def wait_out(c, par):
            pltpu.make_async_copy(out_v[par], out_hbm.at[c, :, pl.ds(b0, BPW)],
                                  osem[par]).wait()

        def transpose(c, par):
            def per_jb(jb):
                jvecd = iota + jb * LANES
                hvec = h_v[c, pl.ds(jb * LANES, LANES)]
                for dblk in range(EMBEDDING_DIM // LANES):
                    for kk in range(LANES):
                        dvec = diak[kk] + dblk * LANES
                        x = plsc.load_gather(rows_v[par], [jvecd, hvec + dvec])
                        plsc.store_scatter(out_v[par], [dvec, jvecd], x)
            pl.loop(0, BPW // LANES)(per_jb)

        start_gather(0, 0)

        def body(t):
            for par in (0, 1):
                c = t * 2 + par

                @pl.when(c + 1 < B_COLS)
                def _():
                    start_gather(c + 1, 1 - par)

                wait_gather(c, par)

                @pl.when(c >= 2)
                def _():
                    wait_out(c - 2, par)

                transpose(c, par)
                start_out(c, par)

        pl.loop(0, B_COLS // 2)(body)
        wait_out(B_COLS - 2, 0)
        wait_out(B_COLS - 1, 1)

    return k(pairs, idx_t)


def kernel(category_indices, embedding_weight):
    idx_t = category_indices.astype(jnp.int32).T
    pairs = embedding_weight.reshape(NUM_CATEGORIES // 2, 128)
    out_t = _gather(pairs, idx_t)
    return out_t.transpose(2, 0, 1)


# speedup vs baseline: 1.2923x; 1.0001x over previous
import functools

import jax
import jax.numpy as jnp
from jax import lax
from jax.experimental import pallas as pl
from jax.experimental.pallas import tpu as pltpu
from jax.experimental.pallas import tpu_sc as plsc

NUM_CATEGORIES = 1000000
EMBEDDING_DIM = 64

NC = 2
NS = 16
NW = NC * NS

B_ROWS = 4096
B_COLS = 100
LANES = 16

BPW = B_ROWS // NW

def _gather(pairs, idx_t):
    mesh = plsc.VectorSubcoreMesh(
        core_axis_name="c", subcore_axis_name="s", num_cores=NC, num_subcores=NS
    )

    @functools.partial(
        pl.kernel,
        out_type=jax.ShapeDtypeStruct((B_COLS, EMBEDDING_DIM, B_ROWS),
                                      jnp.float32),
        mesh=mesh,
        scratch_types=[
            pltpu.VMEM((B_COLS, BPW), jnp.int32),
            pltpu.VMEM((B_COLS, BPW), jnp.int32),
            pltpu.VMEM((B_COLS, BPW), jnp.int32),
            pltpu.VMEM((BPW, 128), jnp.float32),
            pltpu.VMEM((BPW, 128), jnp.float32),
            pltpu.VMEM((EMBEDDING_DIM, BPW), jnp.float32),
            pltpu.VMEM((EMBEDDING_DIM, BPW), jnp.float32),
            pltpu.SemaphoreType.DMA,
            pltpu.SemaphoreType.DMA,
            pltpu.SemaphoreType.DMA,
            pltpu.SemaphoreType.DMA,
        ],
        compiler_params=pltpu.CompilerParams(needs_layout_passes=False),
    )
    def k(scr_hbm, idx_hbm, out_hbm, idx_v, q_v, h_v,
          r0v, r1v, o0v, o1v, gs0, gs1, os0, os1):
        wid = lax.axis_index("s") * NC + lax.axis_index("c")
        b0 = pl.multiple_of(wid * BPW, BPW)
        iota = lax.broadcasted_iota(jnp.int32, (LANES,), 0)
        rows_v = (r0v, r1v)
        out_v = (o0v, o1v)
        gsem = (gs0, gs1)
        osem = (os0, os1)
        jvec = [jb * LANES + iota for jb in range(BPW // LANES)]
        diak = [(iota + kk) & 15 for kk in range(LANES)]

        pltpu.sync_copy(idx_hbm.at[:, pl.ds(b0, BPW)], idx_v)

        def prep(c):
            for j in range(BPW // LANES):
                iv = idx_v[c, pl.ds(j * LANES, LANES)]
                q_v[c, pl.ds(j * LANES, LANES)] = iv >> 1
                h_v[c, pl.ds(j * LANES, LANES)] = (iv & 1) << 6
        pl.loop(0, B_COLS)(prep)

        def start_gather(c, par):
            pltpu.async_copy(scr_hbm.at[q_v.at[c]], rows_v[par], gsem[par])

        def wait_gather(c, par):
            pltpu.make_async_copy(scr_hbm.at[q_v.at[c]], rows_v[par],
                                  gsem[par]).wait()

        def start_out(c, par):
            pltpu.async_copy(out_v[par], out_hbm.at[c, :, pl.ds(b0, BPW)],
                             osem[par])

        def wait_out(c, par):
            pltpu.make_async_copy(out_v[par], out_hbm.at[c, :, pl.ds(b0, BPW)],
                                  osem[par]).wait()

        def transpose(c, par):
            def per_jb(jb):
                jvecd = iota + jb * LANES
                hvec = h_v[c, pl.ds(jb * LANES, LANES)]
                for dblk in range(EMBEDDING_DIM // LANES):
                    for kk in range(LANES):
                        dvec = diak[kk] + dblk * LANES
                        x = plsc.load_gather(rows_v[par], [jvecd, hvec + dvec])
                        plsc.store_scatter(out_v[par], [dvec, jvecd], x)
            pl.loop(0, BPW // LANES)(per_jb)

        start_gather(0, 0)

        def body(t):
            for par in (0, 1):
                c = t * 2 + par

                @pl.when(c + 1 < B_COLS)
                def _():
                    start_gather(c + 1, 1 - par)

                wait_gather(c, par)

                @pl.when(c >= 2)
                def _():
                    wait_out(c - 2, par)

                transpose(c, par)
                start_out(c, par)

        pl.loop(0, B_COLS // 2)(body)
        wait_out(B_COLS - 2, 0)
        wait_out(B_COLS - 1, 1)

    return k(pairs, idx_t)


def kernel(category_indices, embedding_weight):
    idx_t = category_indices.astype(jnp.int32).T
    pairs = embedding_weight.reshape(NUM_CATEGORIES // 2, 128)
    out_t = _gather(pairs, idx_t)
    return out_t.transpose(2, 0, 1)
